# baseline (speedup 1.0000x reference)
import functools

import jax
import jax.numpy as jnp
from jax import lax
from jax.experimental import pallas as pl
from jax.experimental.pallas import tpu as pltpu
from jax.experimental.pallas import tpu_sc as plsc

B = 4096
R = 200
J = 100
D = 64
NC = 2
NS = 16
NW = NC * NS
CH = 4
NCH = J * D // CH // NW
NBUF = 7


def _gather_body(x_hbm, ind_hbm, out_hbm, ind_v, idx_v, rows_v, gsems, osems):
    wid = lax.axis_index("c") * NS + lax.axis_index("s")
    base_m = wid * NCH
    pltpu.sync_copy(ind_hbm, ind_v.at[pl.ds(0, J)])

    def compute_idx(c, carry):
        m = base_m + c
        j = lax.shift_right_logical(m, 4)
        idx_v[pl.ds(c * 16, 16)] = ind_v[pl.ds(j, 16)] * (D // CH) + (m & (D // CH - 1))
        return carry

    lax.fori_loop(0, NCH, compute_idx, 0)

    def fire_gather(c, buf):
        pltpu.async_copy(
            x_hbm.at[idx_v.at[pl.ds(c * 16, 1)]], rows_v.at[buf], gsems[buf])

    def drain_gather(buf):
        pltpu.make_async_copy(
            out_hbm.at[pl.ds(0, 1)], rows_v.at[buf], gsems[buf]).wait()

    def fire_out(c, buf):
        pltpu.async_copy(
            rows_v.at[buf], out_hbm.at[pl.ds(base_m + c, 1)], osems[buf])

    def drain_out(buf):
        pltpu.make_async_copy(
            out_hbm.at[pl.ds(0, 1)], rows_v.at[buf], osems[buf]).wait()

    def step(c, buf):
        @pl.when(c >= NBUF)
        def _():
            drain_out(buf)
        fire_gather(c, buf)
        @pl.when(c >= 1)
        def _():
            drain_gather((buf - 1) % NBUF)
            fire_out(c - 1, (buf - 1) % NBUF)

    def loop(h, carry):
        for buf in range(NBUF):
            step(h * NBUF + buf, buf)
        return carry

    lax.fori_loop(0, (NCH - 1) // NBUF, loop, 0)
    for c in range((NCH - 1) // NBUF * NBUF, NCH):
        step(c, c % NBUF)
    last = (NCH - 1) % NBUF
    drain_gather(last)
    fire_out(NCH - 1, last)
    for buf in range(NBUF):
        drain_out(buf)


@jax.jit
def kernel(x, indices):
    x_t = x.transpose(1, 2, 0).reshape(R * D // CH, CH, B)
    mesh = plsc.VectorSubcoreMesh(core_axis_name="c", subcore_axis_name="s")
    run = functools.partial(
        pl.kernel,
        mesh=mesh,
        out_type=jax.ShapeDtypeStruct((J * D // CH, CH, B), jnp.float32),
        scratch_types=[
            pltpu.VMEM((J + 28,), jnp.int32),
            pltpu.VMEM((NCH * 16,), jnp.int32),
            pltpu.VMEM((NBUF, 1, CH, B), jnp.float32),
            [pltpu.SemaphoreType.DMA] * NBUF,
            [pltpu.SemaphoreType.DMA] * NBUF,
        ],
        compiler_params=pltpu.CompilerParams(use_tc_tiling_on_sc=True),
    )(_gather_body)
    out_t = run(x_t, indices)
    return out_t.reshape(J, D, B).transpose(2, 0, 1)
